# baseline (speedup 1.0000x reference)
import functools

import jax
import jax.numpy as jnp
from jax import lax
from jax.experimental import pallas as pl
from jax.experimental.pallas import tpu as pltpu
from jax.experimental.pallas import tpu_sc as plsc

V = 1000
D = 1024
B = 4
S = 8192

NC = 2
NS = 16
NW = NC * NS
TOK = B * S
TOK_PER_W = TOK // NW
C = 64
NCHUNK = TOK_PER_W // C


def _proj_body(e_ref, w_ref, b_ref, o_ref):
    o_ref[...] = lax.dot_general(
        e_ref[...], w_ref[...],
        dimension_numbers=(((1,), (1,)), ((), ())),
        preferred_element_type=jnp.float32,
        precision=lax.Precision.HIGHEST,
    ) + b_ref[...]


def _proj(embed, W, b2d):
    return pl.pallas_call(
        _proj_body,
        out_shape=jax.ShapeDtypeStruct((V, V), jnp.float32),
    )(embed, W, b2d)


@functools.partial(
    pl.kernel,
    mesh=plsc.VectorSubcoreMesh(core_axis_name="c", subcore_axis_name="s"),
    compiler_params=pltpu.CompilerParams(use_tc_tiling_on_sc=False),
    out_type=jax.ShapeDtypeStruct((TOK, V), jnp.float32),
    scratch_types=[
        pltpu.VMEM((NCHUNK, C), jnp.int32),
        pltpu.VMEM((C, V), jnp.float32),
        pltpu.VMEM((C, V), jnp.float32),
        pltpu.SemaphoreType.DMA,
        pltpu.SemaphoreType.DMA,
        pltpu.SemaphoreType.DMA,
        pltpu.SemaphoreType.DMA,
    ],
)
def _gather(x_hbm, p_hbm, out_hbm, idx_v, buf0, buf1, g0, g1, o0, o1):
    wid = lax.axis_index("s") * NC + lax.axis_index("c")
    base = wid * TOK_PER_W
    pltpu.sync_copy(x_hbm.at[wid], idx_v)

    bufs = (buf0, buf1)
    gsems = (g0, g1)
    osems = (o0, o1)

    gh = [
        pltpu.async_copy(p_hbm.at[idx_v.at[0]], buf0, g0),
        pltpu.async_copy(p_hbm.at[idx_v.at[1]], buf1, g1),
    ]
    for j in range(NCHUNK):
        t = j % 2
        gh[t].wait()
        oh = pltpu.async_copy(bufs[t], out_hbm.at[pl.ds(base + j * C, C)],
                              osems[t])
        oh.wait()
        if j + 2 < NCHUNK:
            gh[t] = pltpu.async_copy(p_hbm.at[idx_v.at[j + 2]], bufs[t],
                                     gsems[t])


def kernel(x, embed, W, b):
    p = _proj(embed, W, b.reshape(1, V))
    xw = x.reshape(NW, NCHUNK, C).astype(jnp.int32)
    out = _gather(xw, p)
    return out.reshape(B, S, V)


# speedup vs baseline: 1.5316x; 1.5316x over previous
import functools

import jax
import jax.numpy as jnp
from jax import lax
from jax.experimental import pallas as pl
from jax.experimental.pallas import tpu as pltpu
from jax.experimental.pallas import tpu_sc as plsc

V = 1000
VP = 1024
VA = 896
VT = V - VA
D = 1024
B = 4
S = 8192

NC = 2
NS = 16
NW = NC * NS
TOK = B * S
TOK_PER_W = TOK // NW
C = 32
NCHUNK = TOK_PER_W // C


def _proj_body(e_ref, w_ref, b_ref, o_ref):
    o_ref[...] = lax.dot_general(
        e_ref[...], w_ref[...],
        dimension_numbers=(((1,), (1,)), ((), ())),
        preferred_element_type=jnp.float32,
        precision=lax.Precision.HIGHEST,
    ) + b_ref[...]


def _proj(embed, W, b2d):
    return pl.pallas_call(
        _proj_body,
        out_shape=jax.ShapeDtypeStruct((V, VP), jnp.float32),
    )(embed, W, b2d)


def _tail_compact(buf, tail, r):
    row_src = buf.at[r]
    row_dst = tail.at[r]
    for w in range(6):
        row_dst[pl.ds(w * 16, 16)] = row_src[pl.ds(VA + w * 16, 16)]
    row_dst[pl.ds(VT - 16, 16)] = row_src[pl.ds(VA + VT - 16, 16)]


@functools.partial(
    pl.kernel,
    mesh=plsc.VectorSubcoreMesh(core_axis_name="c", subcore_axis_name="s"),
    out_type=jax.ShapeDtypeStruct((TOK, V), jnp.float32),
    scratch_types=[
        pltpu.VMEM((NCHUNK, C), jnp.int32),
        pltpu.VMEM((C, VP), jnp.float32),
        pltpu.VMEM((C, VP), jnp.float32),
        pltpu.VMEM((C, VT), jnp.float32),
        pltpu.VMEM((C, VT), jnp.float32),
        pltpu.SemaphoreType.DMA,
        pltpu.SemaphoreType.DMA,
        pltpu.SemaphoreType.DMA,
        pltpu.SemaphoreType.DMA,
    ],
)
def _gather(x_hbm, p_hbm, out_hbm, idx_v, buf0, buf1, tail0, tail1,
            g0, g1, o0, o1):
    wid = lax.axis_index("s") * NC + lax.axis_index("c")
    base = wid * TOK_PER_W
    pltpu.sync_copy(x_hbm.at[wid], idx_v)

    bufs = (buf0, buf1)
    tails = (tail0, tail1)
    gsems = (g0, g1)
    osems = (o0, o1)

    gh = [
        pltpu.async_copy(p_hbm.at[idx_v.at[0]], buf0, g0),
        pltpu.async_copy(p_hbm.at[idx_v.at[1]], buf1, g1),
    ]
    for j in range(NCHUNK):
        t = j % 2
        gh[t].wait()

        def body(r, carry, _t=t):
            _tail_compact(bufs[_t], tails[_t], r)
            return carry

        lax.fori_loop(0, C, body, 0)

        rows = pl.ds(base + j * C, C)
        oa = pltpu.async_copy(bufs[t].at[:, pl.ds(0, VA)],
                              out_hbm.at[rows, pl.ds(0, VA)], osems[t])
        ob = pltpu.async_copy(tails[t], out_hbm.at[rows, pl.ds(VA, VT)],
                              osems[t])
        oa.wait()
        ob.wait()
        if j + 2 < NCHUNK:
            gh[t] = pltpu.async_copy(p_hbm.at[idx_v.at[j + 2]], bufs[t],
                                     gsems[t])


def kernel(x, embed, W, b):
    w_pad = jnp.zeros((VP, D), jnp.float32).at[:V].set(W)
    b_pad = jnp.zeros((1, VP), jnp.float32).at[:, :V].set(b)
    p = _proj(embed, w_pad, b_pad)
    xw = x.reshape(NW, NCHUNK, C).astype(jnp.int32)
    out = _gather(xw, p)
    return out.reshape(B, S, V)
